# Initial kernel scaffold; baseline (speedup 1.0000x reference)
import jax
import jax.numpy as jnp
from jax.experimental import pallas as pl


def kernel(x, edge_index, edge_attr, W1, root1, b1, W2, root2, b2):
    raise NotImplementedError("write your pallas kernel here")


# speedup vs baseline: 2.3724x; 2.3724x over previous
import functools

import jax
import jax.numpy as jnp
from jax import lax
from jax.experimental import pallas as pl
from jax.experimental.pallas import tpu as pltpu
from jax.experimental.pallas import tpu_sc as plsc

N = 10000
E = 160000
D_IN = 256
HID = 16
N_CLS = 40

NC = 2
NS = 16
NW = NC * NS
NP = 10240
RPT = NP // NS
PER_TILE = 5120
EP = PER_TILE * NW
CH = 128
NCH = PER_TILE // CH
W = 128

BLK = 400
GRID = N // BLK


def _axis_index(name):
  return lax.axis_index(name)


def _scatter_add(shared, idx_ref, msg_ref):
  pltpu.sync_copy(msg_ref, shared.at[idx_ref], add=True)


def _make_sc_pass(half: int, with_count: bool):
  mesh = plsc.VectorSubcoreMesh(core_axis_name="c", subcore_axis_name="s",
                                num_cores=NC, num_subcores=NS)

  @functools.partial(
      pl.kernel,
      out_type=jax.ShapeDtypeStruct((2 * NP, W), jnp.float32),
      mesh=mesh,
      scratch_types=[
          pltpu.VMEM((CH,), jnp.int32),
          pltpu.VMEM((CH,), jnp.int32),
          pltpu.VMEM((CH * 16,), jnp.float32),
          pltpu.VMEM((CH, W), jnp.float32),
          pltpu.VMEM((CH, W), jnp.float32),
          pltpu.VMEM_SHARED((NP, W), jnp.float32),
          pltpu.SemaphoreType.DMA,
      ],
  )
  def sc_pass(table_hbm, src_hbm, dst_hbm, usp_hbm, zeros_hbm, out_hbm,
              src_v, dst_v, u_v, rows_v, msg_v, shared, sem):
    cid = _axis_index("c")
    sid = _axis_index("s")
    wid = cid * NS + sid

    pltpu.sync_copy(zeros_hbm.at[pl.ds(sid * RPT, RPT)],
                    shared.at[pl.ds(sid * RPT, RPT)])

    cvec = jnp.where(lax.iota(jnp.int32, 16) == 0, 1.0, 0.0)
    zvec = jnp.zeros((16,), jnp.float32)
    first_const = half // 16

    def init_body(e, carry):
      for c in range(first_const, W // 16):
        if with_count and c == first_const:
          msg_v[e, c * 16:(c + 1) * 16] = cvec
        else:
          msg_v[e, c * 16:(c + 1) * 16] = zvec
      return carry
    lax.fori_loop(0, CH, init_body, 0)

    plsc.subcore_barrier()

    def chunk_body(j, carry):
      widj = wid * NCH + j
      pltpu.sync_copy(src_hbm.at[widj], src_v)
      pltpu.sync_copy(dst_hbm.at[widj], dst_v)
      pltpu.sync_copy(usp_hbm.at[pl.ds(widj * (CH * 16), CH * 16)], u_v)
      pltpu.async_copy(table_hbm.at[src_v], rows_v, sem).wait()

      for e in range(CH):
        uv = u_v[e * 16:(e + 1) * 16]
        for k in range(0, half, 16):
          r0 = rows_v[e, k:k + 16]
          r1 = rows_v[e, half + k:half + k + 16]
          msg_v[e, k:k + 16] = r0 + uv * (r1 - r0)

      _scatter_add(shared, dst_v, msg_v)
      return carry
    lax.fori_loop(0, NCH, chunk_body, 0)

    plsc.subcore_barrier()
    pltpu.sync_copy(shared.at[pl.ds(sid * RPT, RPT)],
                    out_hbm.at[pl.ds(cid * NP + sid * RPT, RPT)])

  return sc_pass


_sc_pass1 = _make_sc_pass(half=HID, with_count=True)
_sc_pass2 = _make_sc_pass(half=48, with_count=False)


def _tc_dense1_body(x_ref, wc_ref, root_ref, b1_ref, y_ref, xr_ref):
  xb = x_ref[...]
  y_ref[...] = jnp.dot(xb, wc_ref[...], preferred_element_type=jnp.float32)
  xr_ref[...] = (jnp.dot(xb, root_ref[...], preferred_element_type=jnp.float32)
                 + b1_ref[...])


def _tc_dense1(x, wc1, root1, b1r):
  return pl.pallas_call(
      _tc_dense1_body,
      grid=(GRID,),
      in_specs=[
          pl.BlockSpec((BLK, D_IN), lambda i: (i, 0)),
          pl.BlockSpec((D_IN, W), lambda i: (0, 0)),
          pl.BlockSpec((D_IN, HID), lambda i: (0, 0)),
          pl.BlockSpec((1, HID), lambda i: (0, 0)),
      ],
      out_specs=[
          pl.BlockSpec((BLK, W), lambda i: (i, 0)),
          pl.BlockSpec((BLK, HID), lambda i: (i, 0)),
      ],
      out_shape=[
          jax.ShapeDtypeStruct((N, W), jnp.float32),
          jax.ShapeDtypeStruct((N, HID), jnp.float32),
      ],
  )(x, wc1, root1, b1r)


def _tc_dense2_body(sc1_ref, xr_ref, wc2_ref, rootp_ref, b2_ref,
                    z_ref, hr_ref):
  tot = sc1_ref[0] + sc1_ref[1]
  cnt = jnp.maximum(tot[:, HID:HID + 1], 1.0)
  h = tot[:, 0:HID] / cnt + xr_ref[...]
  h = jnp.where(h > 0, h, jnp.exp(h) - 1.0)
  z_ref[...] = jnp.dot(h, wc2_ref[...], preferred_element_type=jnp.float32)
  hr_ref[...] = (jnp.dot(h, rootp_ref[...], preferred_element_type=jnp.float32)
                 + b2_ref[...])


def _tc_dense2(sc1, xr, wc2, rootp, b2r):
  return pl.pallas_call(
      _tc_dense2_body,
      grid=(GRID,),
      in_specs=[
          pl.BlockSpec((2, BLK, W), lambda i: (0, i, 0)),
          pl.BlockSpec((BLK, HID), lambda i: (i, 0)),
          pl.BlockSpec((HID, W), lambda i: (0, 0)),
          pl.BlockSpec((HID, 48), lambda i: (0, 0)),
          pl.BlockSpec((1, 48), lambda i: (0, 0)),
      ],
      out_specs=[
          pl.BlockSpec((BLK, W), lambda i: (i, 0)),
          pl.BlockSpec((BLK, 48), lambda i: (i, 0)),
      ],
      out_shape=[
          jax.ShapeDtypeStruct((N, W), jnp.float32),
          jax.ShapeDtypeStruct((N, 48), jnp.float32),
      ],
  )(sc1, xr, wc2, rootp, b2r)


def _tc_final_body(sc2_ref, sc1_ref, hr_ref, out_ref):
  cnt = jnp.maximum(sc1_ref[0][:, HID:HID + 1] + sc1_ref[1][:, HID:HID + 1],
                    1.0)
  logits = (sc2_ref[0][:, 0:48] + sc2_ref[1][:, 0:48]) / cnt + hr_ref[...]
  col = lax.broadcasted_iota(jnp.int32, (BLK, 48), 1)
  logits = jnp.where(col < N_CLS, logits, -1e30)
  m = jnp.max(logits, axis=1, keepdims=True)
  lse = jnp.log(jnp.sum(jnp.exp(logits - m), axis=1, keepdims=True))
  res = logits - m - lse
  out_ref[...] = res[:, 0:N_CLS]


def _tc_final(sc2, sc1, hr):
  return pl.pallas_call(
      _tc_final_body,
      grid=(GRID,),
      in_specs=[
          pl.BlockSpec((2, BLK, W), lambda i: (0, i, 0)),
          pl.BlockSpec((2, BLK, W), lambda i: (0, i, 0)),
          pl.BlockSpec((BLK, 48), lambda i: (i, 0)),
      ],
      out_specs=pl.BlockSpec((BLK, N_CLS), lambda i: (i, 0)),
      out_shape=jax.ShapeDtypeStruct((N, N_CLS), jnp.float32),
  )(sc2, sc1, hr)


def kernel(x, edge_index, edge_attr, W1, root1, b1, W2, root2, b2):
  src = edge_index[0]
  dst = edge_index[1]
  u = edge_attr[:, 0]

  pad = EP - E
  srcp = jnp.concatenate([src, jnp.zeros((pad,), jnp.int32)]
                         ).reshape(NW * NCH, CH)
  dstp = jnp.concatenate([dst, jnp.full((pad,), N, jnp.int32)]
                         ).reshape(NW * NCH, CH)
  up = jnp.concatenate([u, jnp.zeros((pad,), jnp.float32)])
  usp = jnp.broadcast_to(up[:, None], (EP, 16)).reshape(EP * 16)

  wc1 = jnp.zeros((D_IN, W), jnp.float32)
  wc1 = wc1.at[:, 0:HID].set(W1[0]).at[:, HID:2 * HID].set(W1[1])
  b1r = b1.reshape(1, HID)
  wc2 = jnp.zeros((HID, W), jnp.float32)
  wc2 = wc2.at[:, 0:N_CLS].set(W2[0]).at[:, 48:48 + N_CLS].set(W2[1])
  rootp = jnp.zeros((HID, 48), jnp.float32).at[:, 0:N_CLS].set(root2)
  b2r = jnp.zeros((1, 48), jnp.float32).at[0, 0:N_CLS].set(b2)

  zeros_w = jnp.zeros((NP, W), jnp.float32)

  y_ext, xr = _tc_dense1(x, wc1, root1, b1r)
  sc1 = _sc_pass1(y_ext, srcp, dstp, usp, zeros_w).reshape(2, NP, W)
  z_ext, hr = _tc_dense2(sc1, xr, wc2, rootp, b2r)
  sc2 = _sc_pass2(z_ext, srcp, dstp, usp, zeros_w).reshape(2, NP, W)
  return _tc_final(sc2, sc1, hr)


# speedup vs baseline: 3.1008x; 1.3070x over previous
import functools

import jax
import jax.numpy as jnp
from jax import lax
from jax.experimental import pallas as pl
from jax.experimental.pallas import tpu as pltpu
from jax.experimental.pallas import tpu_sc as plsc

N = 10000
E = 160000
D_IN = 256
HID = 16
N_CLS = 40

NC = 2
NS = 16
NW = NC * NS
NP = 10240
RPT = NP // NS
PER_TILE = 5120
EP = PER_TILE * NW
CH = 80
NCH = PER_TILE // CH
W = 128

BLK = 400
GRID = N // BLK


def _axis_index(name):
  return lax.axis_index(name)


def _scatter_add(shared, idx_ref, msg_ref):
  pltpu.sync_copy(msg_ref, shared.at[idx_ref], add=True)


def _make_sc_pass(half: int, aw: int, with_count: bool):
  mesh = plsc.VectorSubcoreMesh(core_axis_name="c", subcore_axis_name="s",
                                num_cores=NC, num_subcores=NS)

  scratch = (
      [pltpu.VMEM((CH,), jnp.int32)] * 4
      + [pltpu.VMEM((CH,), jnp.int32)] * 4
      + [pltpu.VMEM((CH * 16,), jnp.float32)] * 4
      + [pltpu.VMEM((CH, W), jnp.float32)] * 2
      + [pltpu.VMEM((CH, aw), jnp.float32)] * 2
      + [pltpu.VMEM_SHARED((NP, aw), jnp.float32)]
      + [pltpu.SemaphoreType.DMA] * 4
      + [pltpu.SemaphoreType.DMA] * 2
      + [pltpu.SemaphoreType.DMA] * 2
  )

  @functools.partial(
      pl.kernel,
      out_type=jax.ShapeDtypeStruct((2 * NP, aw), jnp.float32),
      mesh=mesh,
      scratch_types=scratch,
  )
  def sc_pass(table_hbm, src_hbm, dst_hbm, usp_hbm, zeros_hbm, out_hbm,
              s0, s1, s2, s3, d0, d1, d2, d3, u0, u1, u2, u3, r0_, r1_,
              m0, m1, shared, qm0, qm1, qm2, qm3, qg0, qg1, qs0, qs1):
    src_v = [s0, s1, s2, s3]
    dst_v = [d0, d1, d2, d3]
    u_v = [u0, u1, u2, u3]
    rows_v = [r0_, r1_]
    msg_v = [m0, m1]
    qm = [qm0, qm1, qm2, qm3]
    qg = [qg0, qg1]
    qs = [qs0, qs1]

    cid = _axis_index("c")
    sid = _axis_index("s")
    wid = cid * NS + sid

    pltpu.sync_copy(zeros_hbm.at[pl.ds(sid * RPT, RPT)],
                    shared.at[pl.ds(sid * RPT, RPT)])

    if with_count:
      cvec = jnp.where(lax.iota(jnp.int32, 16) == 0, 1.0, 0.0)

      def init_body(e, carry):
        for t in range(2):
          msg_v[t][e, half:half + 16] = cvec
        return carry
      lax.fori_loop(0, CH, init_body, 0)

    plsc.subcore_barrier()

    def start_meta(j):
      t = j % 4
      widj = wid * NCH + j
      return (
          pltpu.async_copy(src_hbm.at[widj], src_v[t], qm[t]),
          pltpu.async_copy(dst_hbm.at[widj], dst_v[t], qm[t]),
          pltpu.async_copy(usp_hbm.at[pl.ds(widj * (CH * 16), CH * 16)],
                           u_v[t], qm[t]),
      )

    def start_gather(j):
      return pltpu.async_copy(table_hbm.at[src_v[j % 4]], rows_v[j % 2],
                              qg[j % 2])


    def compute_chunk(j):
      rows, u, msg = rows_v[j % 2], u_v[j % 4], msg_v[j % 2]

      def body(i, carry):
        for s in range(4):
          e = i * 4 + s
          uv = u[pl.ds(e * 16, 16)]
          for k in range(0, half, 16):
            a = rows[e, k:k + 16]
            b = rows[e, half + k:half + k + 16]
            msg[e, k:k + 16] = a + uv * (b - a)
        return carry
      lax.fori_loop(0, CH // 4, body, 0)

    meta = {}
    gat = {}
    scat = {}
    meta[0] = start_meta(0)
    meta[1] = start_meta(1)
    for c in meta[0]:
      c.wait()
    gat[0] = start_gather(0)

    for j in range(NCH):
      if j + 1 < NCH:
        for c in meta[j + 1]:
          c.wait()
        gat[j + 1] = start_gather(j + 1)
      gat[j].wait()
      if j >= 2:
        scat[j - 2].wait()
      compute_chunk(j)
      scat[j] = pltpu.async_copy(msg_v[j % 2], shared.at[dst_v[j % 4]],
                                 qs[j % 2], add=True)
      if j + 2 < NCH:
        meta[j + 2] = start_meta(j + 2)

    for j in range(NCH - 2, NCH):
      scat[j].wait()

    plsc.subcore_barrier()
    pltpu.sync_copy(shared.at[pl.ds(sid * RPT, RPT)],
                    out_hbm.at[pl.ds(cid * NP + sid * RPT, RPT)])

  return sc_pass


AW1 = 128
AW2 = 128
_sc_pass1 = _make_sc_pass(half=HID, aw=AW1, with_count=True)
_sc_pass2 = _make_sc_pass(half=48, aw=AW2, with_count=False)


def _tc_dense1_body(x_ref, wc_ref, root_ref, b1_ref, y_ref, xr_ref):
  xb = x_ref[...]
  y_ref[...] = jnp.dot(xb, wc_ref[...], preferred_element_type=jnp.float32)
  xr_ref[...] = (jnp.dot(xb, root_ref[...], preferred_element_type=jnp.float32)
                 + b1_ref[...])


def _tc_dense1(x, wc1, root1, b1r):
  return pl.pallas_call(
      _tc_dense1_body,
      grid=(GRID,),
      in_specs=[
          pl.BlockSpec((BLK, D_IN), lambda i: (i, 0)),
          pl.BlockSpec((D_IN, W), lambda i: (0, 0)),
          pl.BlockSpec((D_IN, HID), lambda i: (0, 0)),
          pl.BlockSpec((1, HID), lambda i: (0, 0)),
      ],
      out_specs=[
          pl.BlockSpec((BLK, W), lambda i: (i, 0)),
          pl.BlockSpec((BLK, HID), lambda i: (i, 0)),
      ],
      out_shape=[
          jax.ShapeDtypeStruct((N, W), jnp.float32),
          jax.ShapeDtypeStruct((N, HID), jnp.float32),
      ],
  )(x, wc1, root1, b1r)


def _tc_dense2_body(sc1_ref, xr_ref, wc2_ref, rootp_ref, b2_ref,
                    z_ref, hr_ref):
  tot = sc1_ref[0] + sc1_ref[1]
  cnt = jnp.maximum(tot[:, HID:HID + 1], 1.0)
  h = tot[:, 0:HID] / cnt + xr_ref[...]
  h = jnp.where(h > 0, h, jnp.exp(h) - 1.0)
  z_ref[...] = jnp.dot(h, wc2_ref[...], preferred_element_type=jnp.float32)
  hr_ref[...] = (jnp.dot(h, rootp_ref[...], preferred_element_type=jnp.float32)
                 + b2_ref[...])


def _tc_dense2(sc1, xr, wc2, rootp, b2r):
  return pl.pallas_call(
      _tc_dense2_body,
      grid=(GRID,),
      in_specs=[
          pl.BlockSpec((2, BLK, AW1), lambda i: (0, i, 0)),
          pl.BlockSpec((BLK, HID), lambda i: (i, 0)),
          pl.BlockSpec((HID, W), lambda i: (0, 0)),
          pl.BlockSpec((HID, 48), lambda i: (0, 0)),
          pl.BlockSpec((1, 48), lambda i: (0, 0)),
      ],
      out_specs=[
          pl.BlockSpec((BLK, W), lambda i: (i, 0)),
          pl.BlockSpec((BLK, 48), lambda i: (i, 0)),
      ],
      out_shape=[
          jax.ShapeDtypeStruct((N, W), jnp.float32),
          jax.ShapeDtypeStruct((N, 48), jnp.float32),
      ],
  )(sc1, xr, wc2, rootp, b2r)


def _tc_final_body(sc2_ref, sc1_ref, hr_ref, out_ref):
  cnt = jnp.maximum(sc1_ref[0][:, HID:HID + 1] + sc1_ref[1][:, HID:HID + 1],
                    1.0)
  logits = (sc2_ref[0][:, 0:48] + sc2_ref[1][:, 0:48]) / cnt + hr_ref[...]
  col = lax.broadcasted_iota(jnp.int32, (BLK, 48), 1)
  logits = jnp.where(col < N_CLS, logits, -1e30)
  m = jnp.max(logits, axis=1, keepdims=True)
  lse = jnp.log(jnp.sum(jnp.exp(logits - m), axis=1, keepdims=True))
  res = logits - m - lse
  out_ref[...] = res[:, 0:N_CLS]


def _tc_final(sc2, sc1, hr):
  return pl.pallas_call(
      _tc_final_body,
      grid=(GRID,),
      in_specs=[
          pl.BlockSpec((2, BLK, AW2), lambda i: (0, i, 0)),
          pl.BlockSpec((2, BLK, AW1), lambda i: (0, i, 0)),
          pl.BlockSpec((BLK, 48), lambda i: (i, 0)),
      ],
      out_specs=pl.BlockSpec((BLK, N_CLS), lambda i: (i, 0)),
      out_shape=jax.ShapeDtypeStruct((N, N_CLS), jnp.float32),
  )(sc2, sc1, hr)


def kernel(x, edge_index, edge_attr, W1, root1, b1, W2, root2, b2):
  src = edge_index[0]
  dst = edge_index[1]
  u = edge_attr[:, 0]

  pad = EP - E
  srcp = jnp.concatenate([src, jnp.zeros((pad,), jnp.int32)]
                         ).reshape(NW * NCH, CH)
  dstp = jnp.concatenate([dst, jnp.full((pad,), N, jnp.int32)]
                         ).reshape(NW * NCH, CH)
  up = jnp.concatenate([u, jnp.zeros((pad,), jnp.float32)])
  usp = jnp.broadcast_to(up[:, None], (EP, 16)).reshape(EP * 16)

  wc1 = jnp.zeros((D_IN, W), jnp.float32)
  wc1 = wc1.at[:, 0:HID].set(W1[0]).at[:, HID:2 * HID].set(W1[1])
  b1r = b1.reshape(1, HID)
  wc2 = jnp.zeros((HID, W), jnp.float32)
  wc2 = wc2.at[:, 0:N_CLS].set(W2[0]).at[:, 48:48 + N_CLS].set(W2[1])
  rootp = jnp.zeros((HID, 48), jnp.float32).at[:, 0:N_CLS].set(root2)
  b2r = jnp.zeros((1, 48), jnp.float32).at[0, 0:N_CLS].set(b2)

  zeros32 = jnp.zeros((NP, AW1), jnp.float32)
  zeros48 = jnp.zeros((NP, AW2), jnp.float32)

  y_ext, xr = _tc_dense1(x, wc1, root1, b1r)
  sc1 = _sc_pass1(y_ext, srcp, dstp, usp, zeros32).reshape(2, NP, AW1)
  z_ext, hr = _tc_dense2(sc1, xr, wc2, rootp, b2r)
  sc2 = _sc_pass2(z_ext, srcp, dstp, usp, zeros48).reshape(2, NP, AW2)
  return _tc_final(sc2, sc1, hr)


# speedup vs baseline: 5.1844x; 1.6720x over previous
import functools

import jax
import jax.numpy as jnp
from jax import lax
from jax.experimental import pallas as pl
from jax.experimental.pallas import tpu as pltpu
from jax.experimental.pallas import tpu_sc as plsc

N = 10000
E = 160000
D_IN = 256
HID = 16
N_CLS = 40

NC = 2
NS = 16
NW = NC * NS
NP = 10240
RPT = NP // NS
PER_TILE = 5120
EP = PER_TILE * NW
CH = 80
NCH = PER_TILE // CH
W = 128

BLK = 400
GRID = N // BLK


def _axis_index(name):
  return lax.axis_index(name)


def _scatter_add(shared, idx_ref, msg_ref):
  pltpu.sync_copy(msg_ref, shared.at[idx_ref], add=True)


def _make_sc_pass(half: int, aw: int, with_count: bool):
  mesh = plsc.VectorSubcoreMesh(core_axis_name="c", subcore_axis_name="s",
                                num_cores=NC, num_subcores=NS)

  scratch = (
      [pltpu.VMEM((CH,), jnp.int32)] * 4
      + [pltpu.VMEM((CH,), jnp.int32)] * 4
      + [pltpu.VMEM((CH * 16,), jnp.float32)] * 4
      + [pltpu.VMEM((CH, W), jnp.float32)] * 2
      + [pltpu.VMEM((CH, aw), jnp.float32)] * 2
      + [pltpu.VMEM_SHARED((NP, aw), jnp.float32)]
      + [pltpu.SemaphoreType.DMA] * 4
      + [pltpu.SemaphoreType.DMA] * 2
      + [pltpu.SemaphoreType.DMA] * 2
  )

  @functools.partial(
      pl.kernel,
      out_type=jax.ShapeDtypeStruct((2 * NP, aw), jnp.float32),
      mesh=mesh,
      scratch_types=scratch,
  )
  def sc_pass(table_hbm, src_hbm, dst_hbm, usp_hbm, zeros_hbm, out_hbm,
              s0, s1, s2, s3, d0, d1, d2, d3, u0, u1, u2, u3, r0_, r1_,
              m0, m1, shared, qm0, qm1, qm2, qm3, qg0, qg1, qs0, qs1):
    src_v = [s0, s1, s2, s3]
    dst_v = [d0, d1, d2, d3]
    u_v = [u0, u1, u2, u3]
    rows_v = [r0_, r1_]
    msg_v = [m0, m1]
    qm = [qm0, qm1, qm2, qm3]
    qg = [qg0, qg1]
    qs = [qs0, qs1]

    cid = _axis_index("c")
    sid = _axis_index("s")
    wid = cid * NS + sid

    pltpu.sync_copy(zeros_hbm.at[pl.ds(sid * RPT, RPT)],
                    shared.at[pl.ds(sid * RPT, RPT)])

    if with_count:
      cvec = jnp.where(lax.iota(jnp.int32, 16) == 0, 1.0, 0.0)

      def init_body(e, carry):
        for t in range(2):
          msg_v[t][e, half:half + 16] = cvec
        return carry
      lax.fori_loop(0, CH, init_body, 0)

    plsc.subcore_barrier()

    def start_meta(j):
      t = j % 4
      widj = wid * NCH + j
      return (
          pltpu.async_copy(src_hbm.at[widj], src_v[t], qm[t]),
          pltpu.async_copy(dst_hbm.at[widj], dst_v[t], qm[t]),
          pltpu.async_copy(usp_hbm.at[pl.ds(widj * (CH * 16), CH * 16)],
                           u_v[t], qm[t]),
      )

    def start_gather(j):
      return pltpu.async_copy(table_hbm.at[src_v[j % 4]], rows_v[j % 2],
                              qg[j % 2])


    def compute_chunk(j):
      rows, u, msg = rows_v[j % 2], u_v[j % 4], msg_v[j % 2]

      def body(i, carry):
        for s in range(4):
          e = i * 4 + s
          uv = u[pl.ds(e * 16, 16)]
          for k in range(0, half, 16):
            a = rows[e, k:k + 16]
            b = rows[e, half + k:half + k + 16]
            msg[e, k:k + 16] = a + uv * (b - a)
        return carry
      lax.fori_loop(0, CH // 4, body, 0)

    meta = {}
    gat = {}
    scat = {}
    meta[0] = start_meta(0)
    meta[1] = start_meta(1)
    for c in meta[0]:
      c.wait()
    gat[0] = start_gather(0)

    for j in range(NCH):
      if j + 1 < NCH:
        for c in meta[j + 1]:
          c.wait()
        gat[j + 1] = start_gather(j + 1)
      gat[j].wait()
      if j >= 2:
        scat[j - 2].wait()
      compute_chunk(j)
      scat[j] = pltpu.async_copy(msg_v[j % 2], shared.at[dst_v[j % 4]],
                                 qs[j % 2], add=True)
      if j + 2 < NCH:
        meta[j + 2] = start_meta(j + 2)

    for j in range(NCH - 2, NCH):
      scat[j].wait()

    plsc.subcore_barrier()
    pltpu.sync_copy(shared.at[pl.ds(sid * RPT, RPT)],
                    out_hbm.at[pl.ds(cid * NP + sid * RPT, RPT)])

  return sc_pass


AW1 = 128
AW2 = 128
_sc_pass1 = _make_sc_pass(half=HID, aw=AW1, with_count=True)
_sc_pass2 = _make_sc_pass(half=48, aw=AW2, with_count=False)


def _tc_dense1_body(x_ref, wc_ref, root_ref, b1_ref, y_ref, xr_ref):
  xb = x_ref[...]
  y_ref[...] = jnp.dot(xb, wc_ref[...], preferred_element_type=jnp.float32)
  xr_ref[...] = (jnp.dot(xb, root_ref[...], preferred_element_type=jnp.float32)
                 + b1_ref[...])


def _tc_dense1(x, wc1, root1, b1r):
  return pl.pallas_call(
      _tc_dense1_body,
      grid=(GRID,),
      in_specs=[
          pl.BlockSpec((BLK, D_IN), lambda i: (i, 0)),
          pl.BlockSpec((D_IN, W), lambda i: (0, 0)),
          pl.BlockSpec((D_IN, HID), lambda i: (0, 0)),
          pl.BlockSpec((1, HID), lambda i: (0, 0)),
      ],
      out_specs=[
          pl.BlockSpec((BLK, W), lambda i: (i, 0)),
          pl.BlockSpec((BLK, HID), lambda i: (i, 0)),
      ],
      out_shape=[
          jax.ShapeDtypeStruct((N, W), jnp.float32),
          jax.ShapeDtypeStruct((N, HID), jnp.float32),
      ],
  )(x, wc1, root1, b1r)


def _tc_dense2_body(sc1_ref, xr_ref, wc2_ref, rootp_ref, b2_ref,
                    z_ref, hr_ref):
  tot = sc1_ref[0] + sc1_ref[1]
  cnt = jnp.maximum(tot[:, HID:HID + 1], 1.0)
  h = tot[:, 0:HID] / cnt + xr_ref[...]
  h = jnp.where(h > 0, h, jnp.exp(h) - 1.0)
  z_ref[...] = jnp.dot(h, wc2_ref[...], preferred_element_type=jnp.float32)
  hr_ref[...] = (jnp.dot(h, rootp_ref[...], preferred_element_type=jnp.float32)
                 + b2_ref[...])


def _tc_dense2(sc1, xr, wc2, rootp, b2r):
  return pl.pallas_call(
      _tc_dense2_body,
      grid=(GRID,),
      in_specs=[
          pl.BlockSpec((2, BLK, AW1), lambda i: (0, i, 0)),
          pl.BlockSpec((BLK, HID), lambda i: (i, 0)),
          pl.BlockSpec((HID, W), lambda i: (0, 0)),
          pl.BlockSpec((HID, 48), lambda i: (0, 0)),
          pl.BlockSpec((1, 48), lambda i: (0, 0)),
      ],
      out_specs=[
          pl.BlockSpec((BLK, W), lambda i: (i, 0)),
          pl.BlockSpec((BLK, 48), lambda i: (i, 0)),
      ],
      out_shape=[
          jax.ShapeDtypeStruct((N, W), jnp.float32),
          jax.ShapeDtypeStruct((N, 48), jnp.float32),
      ],
  )(sc1, xr, wc2, rootp, b2r)


def _tc_final_body(sc2_ref, sc1_ref, hr_ref, out_ref):
  cnt = jnp.maximum(sc1_ref[0][:, HID:HID + 1] + sc1_ref[1][:, HID:HID + 1],
                    1.0)
  logits = (sc2_ref[0][:, 0:48] + sc2_ref[1][:, 0:48]) / cnt + hr_ref[...]
  col = lax.broadcasted_iota(jnp.int32, (BLK, 48), 1)
  logits = jnp.where(col < N_CLS, logits, -1e30)
  m = jnp.max(logits, axis=1, keepdims=True)
  lse = jnp.log(jnp.sum(jnp.exp(logits - m), axis=1, keepdims=True))
  res = logits - m - lse
  out_ref[...] = res[:, 0:N_CLS]


def _tc_final(sc2, sc1, hr):
  return pl.pallas_call(
      _tc_final_body,
      grid=(GRID,),
      in_specs=[
          pl.BlockSpec((2, BLK, AW2), lambda i: (0, i, 0)),
          pl.BlockSpec((2, BLK, AW1), lambda i: (0, i, 0)),
          pl.BlockSpec((BLK, 48), lambda i: (i, 0)),
      ],
      out_specs=pl.BlockSpec((BLK, N_CLS), lambda i: (i, 0)),
      out_shape=jax.ShapeDtypeStruct((N, N_CLS), jnp.float32),
  )(sc2, sc1, hr)


def kernel(x, edge_index, edge_attr, W1, root1, b1, W2, root2, b2):
  src = edge_index[0]
  dst = edge_index[1]
  u = edge_attr[:, 0]

  pad = EP - E
  pidx = jnp.arange(pad, dtype=jnp.int32)
  srcp = jnp.concatenate([src, pidx % N]).reshape(NW * NCH, CH)
  dstp = jnp.concatenate([dst, N + pidx % (NP - N)]).reshape(NW * NCH, CH)
  up = jnp.concatenate([u, jnp.zeros((pad,), jnp.float32)])
  usp = jnp.broadcast_to(up[:, None], (EP, 16)).reshape(EP * 16)

  wc1 = jnp.zeros((D_IN, W), jnp.float32)
  wc1 = wc1.at[:, 0:HID].set(W1[0]).at[:, HID:2 * HID].set(W1[1])
  b1r = b1.reshape(1, HID)
  wc2 = jnp.zeros((HID, W), jnp.float32)
  wc2 = wc2.at[:, 0:N_CLS].set(W2[0]).at[:, 48:48 + N_CLS].set(W2[1])
  rootp = jnp.zeros((HID, 48), jnp.float32).at[:, 0:N_CLS].set(root2)
  b2r = jnp.zeros((1, 48), jnp.float32).at[0, 0:N_CLS].set(b2)

  zeros32 = jnp.zeros((NP, AW1), jnp.float32)
  zeros48 = jnp.zeros((NP, AW2), jnp.float32)

  y_ext, xr = _tc_dense1(x, wc1, root1, b1r)
  sc1 = _sc_pass1(y_ext, srcp, dstp, usp, zeros32).reshape(2, NP, AW1)
  z_ext, hr = _tc_dense2(sc1, xr, wc2, rootp, b2r)
  sc2 = _sc_pass2(z_ext, srcp, dstp, usp, zeros48).reshape(2, NP, AW2)
  return _tc_final(sc2, sc1, hr)


# speedup vs baseline: 7.2947x; 1.4070x over previous
import functools

import jax
import jax.numpy as jnp
from jax import lax
from jax.experimental import pallas as pl
from jax.experimental.pallas import tpu as pltpu
from jax.experimental.pallas import tpu_sc as plsc

N = 10000
E = 160000
D_IN = 256
HID = 16
N_CLS = 40

NC = 2
NS = 16
NW = NC * NS
NP = 10240
RPT = NP // NS
PER_TILE = 5120
EP = PER_TILE * NW
CH = 80
NCH = PER_TILE // CH
W = 128

BLK = 400
GRID = N // BLK


def _axis_index(name):
  return lax.axis_index(name)


def _scatter_add(shared, idx_ref, msg_ref):
  pltpu.sync_copy(msg_ref, shared.at[idx_ref], add=True)


def _make_sc_pass(half: int, aw: int, with_count: bool):
  mesh = plsc.VectorSubcoreMesh(core_axis_name="c", subcore_axis_name="s",
                                num_cores=NC, num_subcores=NS)

  scratch = (
      [pltpu.VMEM((CH,), jnp.int32)] * 4
      + [pltpu.VMEM((CH,), jnp.int32)] * 4
      + [pltpu.VMEM((128,), jnp.float32)] * 4
      + [pltpu.VMEM((CH, W), jnp.float32)] * 2
      + [pltpu.VMEM((CH, aw), jnp.float32)] * 2
      + [pltpu.VMEM_SHARED((NP, aw), jnp.float32)]
      + [pltpu.SemaphoreType.DMA] * 4
      + [pltpu.SemaphoreType.DMA] * 2
      + [pltpu.SemaphoreType.DMA] * 2
  )

  @functools.partial(
      pl.kernel,
      out_type=jax.ShapeDtypeStruct((2 * NP, aw), jnp.float32),
      mesh=mesh,
      scratch_types=scratch,
  )
  def sc_pass(table_hbm, src_hbm, dst_hbm, usp_hbm, zeros_hbm, out_hbm,
              s0, s1, s2, s3, d0, d1, d2, d3, u0, u1, u2, u3, r0_, r1_,
              m0, m1, shared, qm0, qm1, qm2, qm3, qg0, qg1, qs0, qs1):
    src_v = [s0, s1, s2, s3]
    dst_v = [d0, d1, d2, d3]
    u_v = [u0, u1, u2, u3]
    rows_v = [r0_, r1_]
    msg_v = [m0, m1]
    qm = [qm0, qm1, qm2, qm3]
    qg = [qg0, qg1]
    qs = [qs0, qs1]

    cid = _axis_index("c")
    sid = _axis_index("s")
    wid = cid * NS + sid

    pltpu.sync_copy(zeros_hbm.at[pl.ds(sid * RPT, RPT)],
                    shared.at[pl.ds(sid * RPT, RPT)])

    if with_count:
      cvec = jnp.where(lax.iota(jnp.int32, 16) == 0, 1.0, 0.0)

      def init_body(e, carry):
        for t in range(2):
          msg_v[t][e, half:half + 16] = cvec
        return carry
      lax.fori_loop(0, CH, init_body, 0)

    plsc.subcore_barrier()

    def start_meta(j):
      t = j % 4
      widj = wid * NCH + j
      return (
          pltpu.async_copy(src_hbm.at[widj], src_v[t], qm[t]),
          pltpu.async_copy(dst_hbm.at[widj], dst_v[t], qm[t]),
          pltpu.async_copy(usp_hbm.at[widj], u_v[t], qm[t]),
      )

    def start_gather(j):
      return pltpu.async_copy(table_hbm.at[src_v[j % 4]], rows_v[j % 2],
                              qg[j % 2])


    def compute_chunk(j):
      rows, u, msg = rows_v[j % 2], u_v[j % 4], msg_v[j % 2]

      def body(i, carry):
        for s in range(4):
          e = i * 4 + s
          uv = jnp.broadcast_to(u[pl.ds(e, 16)][0], (16,))
          for k in range(0, half, 16):
            a = rows[e, k:k + 16]
            d = rows[e, half + k:half + k + 16]
            msg[e, k:k + 16] = a + uv * d
        return carry
      lax.fori_loop(0, CH // 4, body, 0)

    meta = {}
    gat = {}
    scat = {}
    meta[0] = start_meta(0)
    meta[1] = start_meta(1)
    for c in meta[0]:
      c.wait()
    gat[0] = start_gather(0)

    for j in range(NCH):
      if j + 1 < NCH:
        for c in meta[j + 1]:
          c.wait()
        gat[j + 1] = start_gather(j + 1)
      gat[j].wait()
      if j >= 2:
        scat[j - 2].wait()
      compute_chunk(j)
      scat[j] = pltpu.async_copy(msg_v[j % 2], shared.at[dst_v[j % 4]],
                                 qs[j % 2], add=True)
      if j + 2 < NCH:
        meta[j + 2] = start_meta(j + 2)

    for j in range(NCH - 2, NCH):
      scat[j].wait()

    plsc.subcore_barrier()
    pltpu.sync_copy(shared.at[pl.ds(sid * RPT, RPT)],
                    out_hbm.at[pl.ds(cid * NP + sid * RPT, RPT)])

  return sc_pass


AW1 = 128
AW2 = 128
_sc_pass1 = _make_sc_pass(half=HID, aw=AW1, with_count=True)
_sc_pass2 = _make_sc_pass(half=48, aw=AW2, with_count=False)


def _tc_dense1_body(x_ref, wc_ref, root_ref, b1_ref, y_ref, xr_ref):
  xb = x_ref[...]
  y_ref[...] = jnp.dot(xb, wc_ref[...], preferred_element_type=jnp.float32)
  xr_ref[...] = (jnp.dot(xb, root_ref[...], preferred_element_type=jnp.float32)
                 + b1_ref[...])


def _tc_dense1(x, wc1, root1, b1r):
  return pl.pallas_call(
      _tc_dense1_body,
      grid=(GRID,),
      in_specs=[
          pl.BlockSpec((BLK, D_IN), lambda i: (i, 0)),
          pl.BlockSpec((D_IN, W), lambda i: (0, 0)),
          pl.BlockSpec((D_IN, HID), lambda i: (0, 0)),
          pl.BlockSpec((1, HID), lambda i: (0, 0)),
      ],
      out_specs=[
          pl.BlockSpec((BLK, W), lambda i: (i, 0)),
          pl.BlockSpec((BLK, HID), lambda i: (i, 0)),
      ],
      out_shape=[
          jax.ShapeDtypeStruct((N, W), jnp.float32),
          jax.ShapeDtypeStruct((N, HID), jnp.float32),
      ],
  )(x, wc1, root1, b1r)


def _tc_dense2_body(sc1_ref, xr_ref, wc2_ref, rootp_ref, b2_ref,
                    z_ref, hr_ref):
  tot = sc1_ref[0] + sc1_ref[1]
  cnt = jnp.maximum(tot[:, HID:HID + 1], 1.0)
  h = tot[:, 0:HID] / cnt + xr_ref[...]
  h = jnp.where(h > 0, h, jnp.exp(h) - 1.0)
  z_ref[...] = jnp.dot(h, wc2_ref[...], preferred_element_type=jnp.float32)
  hr_ref[...] = (jnp.dot(h, rootp_ref[...], preferred_element_type=jnp.float32)
                 + b2_ref[...])


def _tc_dense2(sc1, xr, wc2, rootp, b2r):
  return pl.pallas_call(
      _tc_dense2_body,
      grid=(GRID,),
      in_specs=[
          pl.BlockSpec((2, BLK, AW1), lambda i: (0, i, 0)),
          pl.BlockSpec((BLK, HID), lambda i: (i, 0)),
          pl.BlockSpec((HID, W), lambda i: (0, 0)),
          pl.BlockSpec((HID, 48), lambda i: (0, 0)),
          pl.BlockSpec((1, 48), lambda i: (0, 0)),
      ],
      out_specs=[
          pl.BlockSpec((BLK, W), lambda i: (i, 0)),
          pl.BlockSpec((BLK, 48), lambda i: (i, 0)),
      ],
      out_shape=[
          jax.ShapeDtypeStruct((N, W), jnp.float32),
          jax.ShapeDtypeStruct((N, 48), jnp.float32),
      ],
  )(sc1, xr, wc2, rootp, b2r)


def _tc_final_body(sc2_ref, sc1_ref, hr_ref, out_ref):
  cnt = jnp.maximum(sc1_ref[0][:, HID:HID + 1] + sc1_ref[1][:, HID:HID + 1],
                    1.0)
  logits = (sc2_ref[0][:, 0:48] + sc2_ref[1][:, 0:48]) / cnt + hr_ref[...]
  col = lax.broadcasted_iota(jnp.int32, (BLK, 48), 1)
  logits = jnp.where(col < N_CLS, logits, -1e30)
  m = jnp.max(logits, axis=1, keepdims=True)
  lse = jnp.log(jnp.sum(jnp.exp(logits - m), axis=1, keepdims=True))
  res = logits - m - lse
  out_ref[...] = res[:, 0:N_CLS]


def _tc_final(sc2, sc1, hr):
  return pl.pallas_call(
      _tc_final_body,
      grid=(GRID,),
      in_specs=[
          pl.BlockSpec((2, BLK, AW2), lambda i: (0, i, 0)),
          pl.BlockSpec((2, BLK, AW1), lambda i: (0, i, 0)),
          pl.BlockSpec((BLK, 48), lambda i: (i, 0)),
      ],
      out_specs=pl.BlockSpec((BLK, N_CLS), lambda i: (i, 0)),
      out_shape=jax.ShapeDtypeStruct((N, N_CLS), jnp.float32),
  )(sc2, sc1, hr)


def kernel(x, edge_index, edge_attr, W1, root1, b1, W2, root2, b2):
  src = edge_index[0]
  dst = edge_index[1]
  u = edge_attr[:, 0]

  pad = EP - E
  pidx = jnp.arange(pad, dtype=jnp.int32)
  srcp = jnp.concatenate([src, pidx % N]).reshape(NW * NCH, CH)
  dstp = jnp.concatenate([dst, N + pidx % (NP - N)]).reshape(NW * NCH, CH)
  usp = jnp.pad(
      jnp.concatenate([u, jnp.zeros((pad,), jnp.float32)]
                      ).reshape(NW * NCH, CH),
      ((0, 0), (0, 128 - CH)))

  wc1 = jnp.zeros((D_IN, W), jnp.float32)
  wc1 = wc1.at[:, 0:HID].set(W1[0]).at[:, HID:2 * HID].set(W1[1] - W1[0])
  b1r = b1.reshape(1, HID)
  wc2 = jnp.zeros((HID, W), jnp.float32)
  wc2 = wc2.at[:, 0:N_CLS].set(W2[0]).at[:, 48:48 + N_CLS].set(W2[1] - W2[0])
  rootp = jnp.zeros((HID, 48), jnp.float32).at[:, 0:N_CLS].set(root2)
  b2r = jnp.zeros((1, 48), jnp.float32).at[0, 0:N_CLS].set(b2)

  zeros32 = jnp.zeros((NP, AW1), jnp.float32)
  zeros48 = jnp.zeros((NP, AW2), jnp.float32)

  y_ext, xr = _tc_dense1(x, wc1, root1, b1r)
  sc1 = _sc_pass1(y_ext, srcp, dstp, usp, zeros32).reshape(2, NP, AW1)
  z_ext, hr = _tc_dense2(sc1, xr, wc2, rootp, b2r)
  sc2 = _sc_pass2(z_ext, srcp, dstp, usp, zeros48).reshape(2, NP, AW2)
  return _tc_final(sc2, sc1, hr)


# speedup vs baseline: 8.0855x; 1.1084x over previous
import functools

import jax
import jax.numpy as jnp
from jax import lax
from jax.experimental import pallas as pl
from jax.experimental.pallas import tpu as pltpu
from jax.experimental.pallas import tpu_sc as plsc

N = 10000
E = 160000
D_IN = 256
HID = 16
N_CLS = 40

NC = 2
NS = 16
NW = NC * NS
NP = 10240
RPT = NP // NS
PER_TILE = 5120
EP = PER_TILE * NW
CH = 80
NCH = PER_TILE // CH
W = 128

BLK = 2000
GRID = N // BLK


def _axis_index(name):
  return lax.axis_index(name)


def _scatter_add(shared, idx_ref, msg_ref):
  pltpu.sync_copy(msg_ref, shared.at[idx_ref], add=True)


def _make_sc_pass(half: int, aw: int, with_count: bool):
  mesh = plsc.VectorSubcoreMesh(core_axis_name="c", subcore_axis_name="s",
                                num_cores=NC, num_subcores=NS)

  scratch = (
      [pltpu.VMEM((CH,), jnp.int32)] * 4
      + [pltpu.VMEM((CH,), jnp.int32)] * 4
      + [pltpu.VMEM((128,), jnp.float32)] * 4
      + [pltpu.VMEM((CH, W), jnp.float32)] * 2
      + [pltpu.VMEM((CH, aw), jnp.float32)] * 2
      + [pltpu.VMEM_SHARED((NP, aw), jnp.float32)]
      + [pltpu.SemaphoreType.DMA] * 4
      + [pltpu.SemaphoreType.DMA] * 2
      + [pltpu.SemaphoreType.DMA] * 2
  )

  @functools.partial(
      pl.kernel,
      out_type=jax.ShapeDtypeStruct((2 * NP, aw), jnp.float32),
      mesh=mesh,
      scratch_types=scratch,
  )
  def sc_pass(table_hbm, src_hbm, dst_hbm, usp_hbm, zeros_hbm, out_hbm,
              s0, s1, s2, s3, d0, d1, d2, d3, u0, u1, u2, u3, r0_, r1_,
              m0, m1, shared, qm0, qm1, qm2, qm3, qg0, qg1, qs0, qs1):
    src_v = [s0, s1, s2, s3]
    dst_v = [d0, d1, d2, d3]
    u_v = [u0, u1, u2, u3]
    rows_v = [r0_, r1_]
    msg_v = [m0, m1]
    qm = [qm0, qm1, qm2, qm3]
    qg = [qg0, qg1]
    qs = [qs0, qs1]

    cid = _axis_index("c")
    sid = _axis_index("s")
    wid = cid * NS + sid

    pltpu.sync_copy(zeros_hbm.at[pl.ds(sid * RPT, RPT)],
                    shared.at[pl.ds(sid * RPT, RPT)])

    if with_count:
      cvec = jnp.where(lax.iota(jnp.int32, 16) == 0, 1.0, 0.0)

      def init_body(e, carry):
        for t in range(2):
          msg_v[t][e, half:half + 16] = cvec
        return carry
      lax.fori_loop(0, CH, init_body, 0)

    plsc.subcore_barrier()

    def start_meta(j):
      t = j % 4
      widj = wid * NCH + j
      return (
          pltpu.async_copy(src_hbm.at[widj], src_v[t], qm[t]),
          pltpu.async_copy(dst_hbm.at[widj], dst_v[t], qm[t]),
          pltpu.async_copy(usp_hbm.at[widj], u_v[t], qm[t]),
      )

    def start_gather(j):
      return pltpu.async_copy(table_hbm.at[src_v[j % 4]], rows_v[j % 2],
                              qg[j % 2])


    def compute_chunk(j):
      rows, u, msg = rows_v[j % 2], u_v[j % 4], msg_v[j % 2]

      def body(i, carry):
        for s in range(4):
          e = i * 4 + s
          uv = jnp.broadcast_to(u[pl.ds(e, 16)][0], (16,))
          for k in range(0, half, 16):
            a = rows[e, k:k + 16]
            d = rows[e, half + k:half + k + 16]
            msg[e, k:k + 16] = a + uv * d
        return carry
      lax.fori_loop(0, CH // 4, body, 0)

    meta = {}
    gat = {}
    scat = {}
    meta[0] = start_meta(0)
    meta[1] = start_meta(1)
    for c in meta[0]:
      c.wait()
    gat[0] = start_gather(0)

    for j in range(NCH):
      if j + 1 < NCH:
        for c in meta[j + 1]:
          c.wait()
        gat[j + 1] = start_gather(j + 1)
      gat[j].wait()
      if j >= 2:
        scat[j - 2].wait()
      compute_chunk(j)
      scat[j] = pltpu.async_copy(msg_v[j % 2], shared.at[dst_v[j % 4]],
                                 qs[j % 2], add=True)
      if j + 2 < NCH:
        meta[j + 2] = start_meta(j + 2)

    for j in range(NCH - 2, NCH):
      scat[j].wait()

    plsc.subcore_barrier()
    pltpu.sync_copy(shared.at[pl.ds(sid * RPT, RPT)],
                    out_hbm.at[pl.ds(cid * NP + sid * RPT, RPT)])

  return sc_pass


AW1 = 128
AW2 = 128
_sc_pass1 = _make_sc_pass(half=HID, aw=AW1, with_count=True)
_sc_pass2 = _make_sc_pass(half=48, aw=AW2, with_count=False)


def _tc_dense1_body(x_ref, wc_ref, root_ref, b1_ref, y_ref, xr_ref):
  xb = x_ref[...]
  y_ref[...] = jnp.dot(xb, wc_ref[...], preferred_element_type=jnp.float32)
  xr_ref[...] = (jnp.dot(xb, root_ref[...], preferred_element_type=jnp.float32)
                 + b1_ref[...])


def _tc_dense1(x, wc1, root1, b1r):
  return pl.pallas_call(
      _tc_dense1_body,
      grid=(GRID,),
      in_specs=[
          pl.BlockSpec((BLK, D_IN), lambda i: (i, 0)),
          pl.BlockSpec((D_IN, W), lambda i: (0, 0)),
          pl.BlockSpec((D_IN, HID), lambda i: (0, 0)),
          pl.BlockSpec((1, HID), lambda i: (0, 0)),
      ],
      out_specs=[
          pl.BlockSpec((BLK, W), lambda i: (i, 0)),
          pl.BlockSpec((BLK, HID), lambda i: (i, 0)),
      ],
      out_shape=[
          jax.ShapeDtypeStruct((N, W), jnp.float32),
          jax.ShapeDtypeStruct((N, HID), jnp.float32),
      ],
  )(x, wc1, root1, b1r)


def _tc_dense2_body(sc1_ref, xr_ref, wc2_ref, rootp_ref, b2_ref,
                    z_ref, hr_ref):
  tot = sc1_ref[0] + sc1_ref[1]
  cnt = jnp.maximum(tot[:, HID:HID + 1], 1.0)
  h = tot[:, 0:HID] / cnt + xr_ref[...]
  h = jnp.where(h > 0, h, jnp.exp(h) - 1.0)
  z_ref[...] = jnp.dot(h, wc2_ref[...], preferred_element_type=jnp.float32)
  hr_ref[...] = (jnp.dot(h, rootp_ref[...], preferred_element_type=jnp.float32)
                 + b2_ref[...])


def _tc_dense2(sc1, xr, wc2, rootp, b2r):
  return pl.pallas_call(
      _tc_dense2_body,
      grid=(GRID,),
      in_specs=[
          pl.BlockSpec((2, BLK, AW1), lambda i: (0, i, 0)),
          pl.BlockSpec((BLK, HID), lambda i: (i, 0)),
          pl.BlockSpec((HID, W), lambda i: (0, 0)),
          pl.BlockSpec((HID, 48), lambda i: (0, 0)),
          pl.BlockSpec((1, 48), lambda i: (0, 0)),
      ],
      out_specs=[
          pl.BlockSpec((BLK, W), lambda i: (i, 0)),
          pl.BlockSpec((BLK, 48), lambda i: (i, 0)),
      ],
      out_shape=[
          jax.ShapeDtypeStruct((N, W), jnp.float32),
          jax.ShapeDtypeStruct((N, 48), jnp.float32),
      ],
  )(sc1, xr, wc2, rootp, b2r)


def _tc_final_body(sc2_ref, sc1_ref, hr_ref, out_ref):
  cnt = jnp.maximum(sc1_ref[0][:, HID:HID + 1] + sc1_ref[1][:, HID:HID + 1],
                    1.0)
  logits = (sc2_ref[0][:, 0:48] + sc2_ref[1][:, 0:48]) / cnt + hr_ref[...]
  col = lax.broadcasted_iota(jnp.int32, (BLK, 48), 1)
  logits = jnp.where(col < N_CLS, logits, -1e30)
  m = jnp.max(logits, axis=1, keepdims=True)
  lse = jnp.log(jnp.sum(jnp.exp(logits - m), axis=1, keepdims=True))
  res = logits - m - lse
  out_ref[...] = res[:, 0:N_CLS]


def _tc_final(sc2, sc1, hr):
  return pl.pallas_call(
      _tc_final_body,
      grid=(GRID,),
      in_specs=[
          pl.BlockSpec((2, BLK, AW2), lambda i: (0, i, 0)),
          pl.BlockSpec((2, BLK, AW1), lambda i: (0, i, 0)),
          pl.BlockSpec((BLK, 48), lambda i: (i, 0)),
      ],
      out_specs=pl.BlockSpec((BLK, N_CLS), lambda i: (i, 0)),
      out_shape=jax.ShapeDtypeStruct((N, N_CLS), jnp.float32),
  )(sc2, sc1, hr)


def kernel(x, edge_index, edge_attr, W1, root1, b1, W2, root2, b2):
  src = edge_index[0]
  dst = edge_index[1]
  u = edge_attr[:, 0]

  pad = EP - E
  pidx = jnp.arange(pad, dtype=jnp.int32)
  srcp = jnp.concatenate([src, pidx % N]).reshape(NW * NCH, CH)
  dstp = jnp.concatenate([dst, N + pidx % (NP - N)]).reshape(NW * NCH, CH)
  usp = jnp.pad(
      jnp.concatenate([u, jnp.zeros((pad,), jnp.float32)]
                      ).reshape(NW * NCH, CH),
      ((0, 0), (0, 128 - CH)))

  wc1 = jnp.zeros((D_IN, W), jnp.float32)
  wc1 = wc1.at[:, 0:HID].set(W1[0]).at[:, HID:2 * HID].set(W1[1] - W1[0])
  b1r = b1.reshape(1, HID)
  wc2 = jnp.zeros((HID, W), jnp.float32)
  wc2 = wc2.at[:, 0:N_CLS].set(W2[0]).at[:, 48:48 + N_CLS].set(W2[1] - W2[0])
  rootp = jnp.zeros((HID, 48), jnp.float32).at[:, 0:N_CLS].set(root2)
  b2r = jnp.zeros((1, 48), jnp.float32).at[0, 0:N_CLS].set(b2)

  zeros32 = jnp.zeros((NP, AW1), jnp.float32)
  zeros48 = jnp.zeros((NP, AW2), jnp.float32)

  y_ext, xr = _tc_dense1(x, wc1, root1, b1r)
  sc1 = _sc_pass1(y_ext, srcp, dstp, usp, zeros32).reshape(2, NP, AW1)
  z_ext, hr = _tc_dense2(sc1, xr, wc2, rootp, b2r)
  sc2 = _sc_pass2(z_ext, srcp, dstp, usp, zeros48).reshape(2, NP, AW2)
  return _tc_final(sc2, sc1, hr)


# speedup vs baseline: 8.0867x; 1.0001x over previous
import functools

import jax
import jax.numpy as jnp
from jax import lax
from jax.experimental import pallas as pl
from jax.experimental.pallas import tpu as pltpu
from jax.experimental.pallas import tpu_sc as plsc

N = 10000
E = 160000
D_IN = 256
HID = 16
N_CLS = 40

NC = 2
NS = 16
NW = NC * NS
NP = 10240
RPT = NP // NS
PER_TILE = 5120
EP = PER_TILE * NW
CH = 80
NCH = PER_TILE // CH
W = 128

BLK = 2000
GRID = N // BLK


def _axis_index(name):
  return lax.axis_index(name)


def _scatter_add(shared, idx_ref, msg_ref):
  pltpu.sync_copy(msg_ref, shared.at[idx_ref], add=True)


def _make_sc_pass(half: int, aw: int, tw: int, with_count: bool):
  mesh = plsc.VectorSubcoreMesh(core_axis_name="c", subcore_axis_name="s",
                                num_cores=NC, num_subcores=NS)

  scratch = (
      [pltpu.VMEM((CH,), jnp.int32)] * 4
      + [pltpu.VMEM((CH,), jnp.int32)] * 4
      + [pltpu.VMEM((128,), jnp.float32)] * 4
      + [pltpu.VMEM((CH, tw), jnp.float32)] * 2
      + [pltpu.VMEM((CH, aw), jnp.float32)] * 2
      + [pltpu.VMEM_SHARED((NP, aw), jnp.float32)]
      + [pltpu.SemaphoreType.DMA] * 4
      + [pltpu.SemaphoreType.DMA] * 2
      + [pltpu.SemaphoreType.DMA] * 2
  )

  @functools.partial(
      pl.kernel,
      out_type=jax.ShapeDtypeStruct((2 * NP, aw), jnp.float32),
      mesh=mesh,
      scratch_types=scratch,
  )
  def sc_pass(table_hbm, src_hbm, dst_hbm, usp_hbm, zeros_hbm, out_hbm,
              s0, s1, s2, s3, d0, d1, d2, d3, u0, u1, u2, u3, r0_, r1_,
              m0, m1, shared, qm0, qm1, qm2, qm3, qg0, qg1, qs0, qs1):
    src_v = [s0, s1, s2, s3]
    dst_v = [d0, d1, d2, d3]
    u_v = [u0, u1, u2, u3]
    rows_v = [r0_, r1_]
    msg_v = [m0, m1]
    qm = [qm0, qm1, qm2, qm3]
    qg = [qg0, qg1]
    qs = [qs0, qs1]

    cid = _axis_index("c")
    sid = _axis_index("s")
    wid = cid * NS + sid

    pltpu.sync_copy(zeros_hbm.at[pl.ds(sid * RPT, RPT)],
                    shared.at[pl.ds(sid * RPT, RPT)])

    if with_count:
      cvec = jnp.where(lax.iota(jnp.int32, 16) == 0, 1.0, 0.0)

      def init_body(e, carry):
        for t in range(2):
          msg_v[t][e, half:half + 16] = cvec
        return carry
      lax.fori_loop(0, CH, init_body, 0)

    plsc.subcore_barrier()

    def start_meta(j):
      t = j % 4
      widj = wid * NCH + j
      return (
          pltpu.async_copy(src_hbm.at[widj], src_v[t], qm[t]),
          pltpu.async_copy(dst_hbm.at[widj], dst_v[t], qm[t]),
          pltpu.async_copy(usp_hbm.at[widj], u_v[t], qm[t]),
      )

    def start_gather(j):
      return pltpu.async_copy(table_hbm.at[src_v[j % 4]], rows_v[j % 2],
                              qg[j % 2])


    def compute_chunk(j):
      rows, u, msg = rows_v[j % 2], u_v[j % 4], msg_v[j % 2]

      def body(i, carry):
        for s in range(4):
          e = i * 4 + s
          uv = jnp.broadcast_to(u[pl.ds(e, 16)][0], (16,))
          for k in range(0, half, 16):
            a = rows[e, k:k + 16]
            d = rows[e, half + k:half + k + 16]
            msg[e, k:k + 16] = a + uv * d
        return carry
      lax.fori_loop(0, CH // 4, body, 0)

    meta = {}
    gat = {}
    scat = {}
    meta[0] = start_meta(0)
    meta[1] = start_meta(1)
    for c in meta[0]:
      c.wait()
    gat[0] = start_gather(0)

    for j in range(NCH):
      if j + 1 < NCH:
        for c in meta[j + 1]:
          c.wait()
        gat[j + 1] = start_gather(j + 1)
      gat[j].wait()
      if j >= 2:
        scat[j - 2].wait()
      compute_chunk(j)
      scat[j] = pltpu.async_copy(msg_v[j % 2], shared.at[dst_v[j % 4]],
                                 qs[j % 2], add=True)
      if j + 2 < NCH:
        meta[j + 2] = start_meta(j + 2)

    for j in range(NCH - 2, NCH):
      scat[j].wait()

    plsc.subcore_barrier()
    pltpu.sync_copy(shared.at[pl.ds(sid * RPT, RPT)],
                    out_hbm.at[pl.ds(cid * NP + sid * RPT, RPT)])

  return sc_pass


AW1 = 128
AW2 = 128
_sc_pass1 = _make_sc_pass(half=HID, aw=AW1, tw=W, with_count=True)
_sc_pass2 = _make_sc_pass(half=48, aw=AW2, tw=W, with_count=False)


def _tc_dense1_body(x_ref, wc_ref, root_ref, b1_ref, y_ref, xr_ref):
  xb = x_ref[...]
  y_ref[...] = jnp.dot(xb, wc_ref[...], preferred_element_type=jnp.float32)
  xr_ref[...] = (jnp.dot(xb, root_ref[...], preferred_element_type=jnp.float32)
                 + b1_ref[...])


def _tc_dense1(x, wc1, root1, b1r):
  return pl.pallas_call(
      _tc_dense1_body,
      grid=(GRID,),
      in_specs=[
          pl.BlockSpec((BLK, D_IN), lambda i: (i, 0)),
          pl.BlockSpec((D_IN, W), lambda i: (0, 0)),
          pl.BlockSpec((D_IN, HID), lambda i: (0, 0)),
          pl.BlockSpec((1, HID), lambda i: (0, 0)),
      ],
      out_specs=[
          pl.BlockSpec((BLK, W), lambda i: (i, 0)),
          pl.BlockSpec((BLK, HID), lambda i: (i, 0)),
      ],
      out_shape=[
          jax.ShapeDtypeStruct((N, W), jnp.float32),
          jax.ShapeDtypeStruct((N, HID), jnp.float32),
      ],
  )(x, wc1, root1, b1r)


def _tc_dense2_body(sc1_ref, xr_ref, wc2_ref, rootp_ref, b2_ref,
                    z_ref, hr_ref):
  tot = sc1_ref[0] + sc1_ref[1]
  cnt = jnp.maximum(tot[:, HID:HID + 1], 1.0)
  h = tot[:, 0:HID] / cnt + xr_ref[...]
  h = jnp.where(h > 0, h, jnp.exp(h) - 1.0)
  z_ref[...] = jnp.dot(h, wc2_ref[...], preferred_element_type=jnp.float32)
  hr_ref[...] = (jnp.dot(h, rootp_ref[...], preferred_element_type=jnp.float32)
                 + b2_ref[...])


def _tc_dense2(sc1, xr, wc2, rootp, b2r):
  return pl.pallas_call(
      _tc_dense2_body,
      grid=(GRID,),
      in_specs=[
          pl.BlockSpec((2, BLK, AW1), lambda i: (0, i, 0)),
          pl.BlockSpec((BLK, HID), lambda i: (i, 0)),
          pl.BlockSpec((HID, W), lambda i: (0, 0)),
          pl.BlockSpec((HID, 48), lambda i: (0, 0)),
          pl.BlockSpec((1, 48), lambda i: (0, 0)),
      ],
      out_specs=[
          pl.BlockSpec((BLK, W), lambda i: (i, 0)),
          pl.BlockSpec((BLK, 48), lambda i: (i, 0)),
      ],
      out_shape=[
          jax.ShapeDtypeStruct((N, W), jnp.float32),
          jax.ShapeDtypeStruct((N, 48), jnp.float32),
      ],
  )(sc1, xr, wc2, rootp, b2r)


def _tc_final_body(sc2_ref, sc1_ref, hr_ref, out_ref):
  cnt = jnp.maximum(sc1_ref[0][:, HID:HID + 1] + sc1_ref[1][:, HID:HID + 1],
                    1.0)
  logits = (sc2_ref[0][:, 0:48] + sc2_ref[1][:, 0:48]) / cnt + hr_ref[...]
  col = lax.broadcasted_iota(jnp.int32, (BLK, 48), 1)
  logits = jnp.where(col < N_CLS, logits, -1e30)
  m = jnp.max(logits, axis=1, keepdims=True)
  lse = jnp.log(jnp.sum(jnp.exp(logits - m), axis=1, keepdims=True))
  res = logits - m - lse
  out_ref[...] = res[:, 0:N_CLS]


def _tc_final(sc2, sc1, hr):
  return pl.pallas_call(
      _tc_final_body,
      grid=(GRID,),
      in_specs=[
          pl.BlockSpec((2, BLK, AW2), lambda i: (0, i, 0)),
          pl.BlockSpec((2, BLK, AW1), lambda i: (0, i, 0)),
          pl.BlockSpec((BLK, 48), lambda i: (i, 0)),
      ],
      out_specs=pl.BlockSpec((BLK, N_CLS), lambda i: (i, 0)),
      out_shape=jax.ShapeDtypeStruct((N, N_CLS), jnp.float32),
  )(sc2, sc1, hr)


def kernel(x, edge_index, edge_attr, W1, root1, b1, W2, root2, b2):
  src = edge_index[0]
  dst = edge_index[1]
  u = edge_attr[:, 0]

  pad = EP - E
  pidx = jnp.arange(pad, dtype=jnp.int32)
  srcp = jnp.concatenate([src, pidx % N]).reshape(NW * NCH, CH)
  dstp = jnp.concatenate([dst, N + pidx % (NP - N)]).reshape(NW * NCH, CH)
  usp = jnp.pad(
      jnp.concatenate([u, jnp.zeros((pad,), jnp.float32)]
                      ).reshape(NW * NCH, CH),
      ((0, 0), (0, 128 - CH)))

  wc1 = jnp.zeros((D_IN, W), jnp.float32)
  wc1 = wc1.at[:, 0:HID].set(W1[0]).at[:, HID:2 * HID].set(W1[1] - W1[0])
  b1r = b1.reshape(1, HID)
  wc2 = jnp.zeros((HID, W), jnp.float32)
  wc2 = wc2.at[:, 0:N_CLS].set(W2[0]).at[:, 48:48 + N_CLS].set(W2[1] - W2[0])
  rootp = jnp.zeros((HID, 48), jnp.float32).at[:, 0:N_CLS].set(root2)
  b2r = jnp.zeros((1, 48), jnp.float32).at[0, 0:N_CLS].set(b2)

  zeros32 = jnp.zeros((NP, AW1), jnp.float32)
  zeros48 = jnp.zeros((NP, AW2), jnp.float32)

  y_ext, xr = _tc_dense1(x, wc1, root1, b1r)
  sc1 = _sc_pass1(y_ext, srcp, dstp, usp, zeros32).reshape(2, NP, AW1)
  z_ext, hr = _tc_dense2(sc1, xr, wc2, rootp, b2r)
  sc2 = _sc_pass2(z_ext, srcp, dstp, usp, zeros48).reshape(2, NP, AW2)
  return _tc_final(sc2, sc1, hr)
